# baseline (speedup 1.0000x reference)
import functools

import jax
import jax.numpy as jnp
from jax import lax
from jax.experimental import pallas as pl
from jax.experimental.pallas import tpu as pltpu
from jax.experimental.pallas import tpu_sc as plsc

DMODEL = 64
NTOK = 16384
NSEQ = 50
B_TOTAL = NTOK * NSEQ

_info = plsc.get_sparse_core_info()
NC, NS = _info.num_cores, _info.num_subcores
NW = NC * NS
BLK = 128
BLKS_PER_W = (NTOK // BLK) // NW
B_PER_W = B_TOTAL // NW
N_UNITS = BLKS_PER_W * NSEQ


@functools.partial(
    pl.kernel,
    mesh=plsc.VectorSubcoreMesh(core_axis_name="c", subcore_axis_name="s"),
    out_type=jax.ShapeDtypeStruct((NSEQ, 8, NTOK // BLK, 8, BLK), jnp.float32),
    scratch_types=[
        pltpu.VMEM((B_PER_W,), jnp.int32),
        pltpu.VMEM((BLK, DMODEL), jnp.float32),
        pltpu.VMEM((BLK, DMODEL), jnp.float32),
        pltpu.VMEM((8, 8, BLK), jnp.float32),
        pltpu.VMEM((8, 8, BLK), jnp.float32),
        pltpu.VMEM((BLK,), jnp.int32),
        pltpu.VMEM((BLK,), jnp.int32),
        pltpu.SemaphoreType.DMA,
        pltpu.SemaphoreType.DMA,
        pltpu.SemaphoreType.DMA,
        pltpu.SemaphoreType.DMA,
    ],
    compiler_params=pltpu.CompilerParams(
        use_tc_tiling_on_sc=False, needs_layout_passes=False),
)
def _embed_sc(idx_hbm, table_hbm, out_hbm, idx_all,
              rows0, rows1, t0, t1, gx0, gx1, sg0, sg1, sw0, sw1):
    rows = (rows0, rows1)
    tb = (t0, t1)
    gx = (gx0, gx1)
    sg = (sg0, sg1)
    sw = (sw0, sw1)
    wid = lax.axis_index("s") * NC + lax.axis_index("c")
    base_blk = wid * BLKS_PER_W

    pltpu.sync_copy(idx_hbm.at[pl.ds(wid * B_PER_W, B_PER_W)], idx_all)

    lane = jax.lax.iota(jnp.int32, 16)
    lane50 = lane * NSEQ

    def build_gidx(b, u):
        s = u // BLKS_PER_W
        lb = u % BLKS_PER_W
        base_p = lb * (BLK * NSEQ) + s
        for g in range(8):
            vals = plsc.load_gather(idx_all, [lane50 + (base_p + g * 16 * NSEQ)])
            gx[b][pl.ds(g * 16, 16)] = vals

    def fire_gather(b):
        pltpu.async_copy(table_hbm.at[gx[b]], rows[b], sg[b])

    def wait_gather(b):
        pltpu.make_async_copy(table_hbm.at[gx[b]], rows[b], sg[b]).wait()

    def transpose(b):
        @plsc.parallel_loop(0, 8, unroll=2)
        def db_body(db):
            d0 = db * 8
            for dr in range(8):
                dv = lane * 0 + (d0 + dr)
                vs = [plsc.load_gather(rows[b], [lane + g * 16, dv])
                      for g in range(8)]
                for g in range(8):
                    tb[b][db, dr, pl.ds(g * 16, 16)] = vs[g]

    def fire_wb(b, u):
        s = u // BLKS_PER_W
        bb = base_blk + u % BLKS_PER_W
        pltpu.async_copy(tb[b], out_hbm.at[s, :, bb], sw[b])

    def wait_wb(b):
        pltpu.make_async_copy(tb[b], out_hbm.at[0, :, 0], sw[b]).wait()

    build_gidx(0, 0)
    fire_gather(0)
    build_gidx(1, 1)
    fire_gather(1)
    for u0 in (0, 1):
        b = u0 % 2
        wait_gather(b)
        transpose(b)
        fire_wb(b, u0)
        build_gidx(b, u0 + 2)
        fire_gather(b)

    def outer(k, carry):
        for i in range(2):
            u = 2 * k + i
            b = i
            wait_gather(b)
            wait_wb(b)
            transpose(b)
            fire_wb(b, u)
            build_gidx(b, u + 2)
            fire_gather(b)
        return carry

    lax.fori_loop(1, N_UNITS // 2 - 1, outer, 0)

    for u0 in (N_UNITS - 2, N_UNITS - 1):
        b = u0 % 2
        wait_gather(b)
        wait_wb(b)
        transpose(b)
        fire_wb(b, u0)
    wait_wb(0)
    wait_wb(1)


def kernel(indices, table):
    idx_flat = indices.reshape(-1).astype(jnp.int32)
    out5 = _embed_sc(idx_flat, table)
    return out5.transpose(2, 4, 0, 1, 3).reshape(NTOK, NSEQ, DMODEL)


# speedup vs baseline: 1.6113x; 1.6113x over previous
import functools

import jax
import jax.numpy as jnp
from jax import lax
from jax.experimental import pallas as pl
from jax.experimental.pallas import tpu as pltpu
from jax.experimental.pallas import tpu_sc as plsc

DMODEL = 64
NTOK = 16384
NSEQ = 50
B_TOTAL = NTOK * NSEQ

_info = plsc.get_sparse_core_info()
NC, NS = _info.num_cores, _info.num_subcores
NW = NC * NS
BLK = 128
BLKS_PER_W = (NTOK // BLK) // NW
B_PER_W = B_TOTAL // NW
N_UNITS = BLKS_PER_W * NSEQ


@functools.partial(
    pl.kernel,
    mesh=plsc.VectorSubcoreMesh(core_axis_name="c", subcore_axis_name="s"),
    out_type=jax.ShapeDtypeStruct((NSEQ, 8, NTOK // BLK, 8, BLK), jnp.float32),
    scratch_types=[
        pltpu.VMEM((B_PER_W,), jnp.int32),
        pltpu.VMEM((BLK, DMODEL), jnp.float32),
        pltpu.VMEM((BLK, DMODEL), jnp.float32),
        pltpu.VMEM((8, 8, BLK), jnp.float32),
        pltpu.VMEM((8, 8, BLK), jnp.float32),
        pltpu.VMEM((BLK,), jnp.int32),
        pltpu.VMEM((BLK,), jnp.int32),
        pltpu.SemaphoreType.DMA,
        pltpu.SemaphoreType.DMA,
        pltpu.SemaphoreType.DMA,
        pltpu.SemaphoreType.DMA,
    ],
    compiler_params=pltpu.CompilerParams(
        use_tc_tiling_on_sc=False, needs_layout_passes=False),
)
def _embed_sc(idx_hbm, table_hbm, out_hbm, idx_all,
              rows0, rows1, t0, t1, gx0, gx1, sg0, sg1, sw0, sw1):
    rows = (rows0, rows1)
    tb = (t0, t1)
    gx = (gx0, gx1)
    sg = (sg0, sg1)
    sw = (sw0, sw1)
    wid = lax.axis_index("s") * NC + lax.axis_index("c")
    base_blk = wid * BLKS_PER_W

    pltpu.sync_copy(idx_hbm.at[pl.ds(wid * B_PER_W, B_PER_W)], idx_all)

    lane = jax.lax.iota(jnp.int32, 16)
    lane50 = lane * NSEQ

    def build_gidx(b, u):
        s = u // BLKS_PER_W
        lb = u % BLKS_PER_W
        base_p = lb * (BLK * NSEQ) + s
        for g in range(8):
            vals = plsc.load_gather(idx_all, [lane50 + (base_p + g * 16 * NSEQ)])
            gx[b][pl.ds(g * 16, 16)] = vals

    def fire_gather(b):
        pltpu.async_copy(table_hbm.at[gx[b]], rows[b], sg[b])

    def wait_gather(b):
        pltpu.make_async_copy(table_hbm.at[gx[b]], rows[b], sg[b]).wait()

    def transpose(b):
        def r_body(r, carry):
            rot = (lane + r) & 15
            rot8 = rot >> 3
            rotm8 = rot & 7
            for k in range(4):
                dvec = rot + k * 16
                i0 = rot8 + 2 * k
                for g in range(8):
                    jvec = lane + g * 16
                    v = plsc.load_gather(rows[b], [jvec, dvec])
                    plsc.store_scatter(tb[b], [i0, rotm8, jvec], v)
            return carry
        lax.fori_loop(0, 16, r_body, 0)

    def fire_wb(b, u):
        s = u // BLKS_PER_W
        bb = base_blk + u % BLKS_PER_W
        pltpu.async_copy(tb[b], out_hbm.at[s, :, bb], sw[b])

    def wait_wb(b):
        pltpu.make_async_copy(tb[b], out_hbm.at[0, :, 0], sw[b]).wait()

    build_gidx(0, 0)
    fire_gather(0)
    build_gidx(1, 1)
    fire_gather(1)
    for u0 in (0, 1):
        b = u0 % 2
        wait_gather(b)
        transpose(b)
        fire_wb(b, u0)
        build_gidx(b, u0 + 2)
        fire_gather(b)

    def outer(k, carry):
        for i in range(2):
            u = 2 * k + i
            b = i
            wait_gather(b)
            wait_wb(b)
            transpose(b)
            fire_wb(b, u)
            build_gidx(b, u + 2)
            fire_gather(b)
        return carry

    lax.fori_loop(1, N_UNITS // 2 - 1, outer, 0)

    for u0 in (N_UNITS - 2, N_UNITS - 1):
        b = u0 % 2
        wait_gather(b)
        wait_wb(b)
        transpose(b)
        fire_wb(b, u0)
    wait_wb(0)
    wait_wb(1)


def kernel(indices, table):
    idx_flat = indices.reshape(-1).astype(jnp.int32)
    out5 = _embed_sc(idx_flat, table)
    return out5.transpose(2, 4, 0, 1, 3).reshape(NTOK, NSEQ, DMODEL)


# speedup vs baseline: 1.9882x; 1.2339x over previous
import functools

import jax
import jax.numpy as jnp
from jax import lax
from jax.experimental import pallas as pl
from jax.experimental.pallas import tpu as pltpu
from jax.experimental.pallas import tpu_sc as plsc

DMODEL = 64
NTOK = 16384
NSEQ = 50
B_TOTAL = NTOK * NSEQ

_info = plsc.get_sparse_core_info()
NC, NS = _info.num_cores, _info.num_subcores
NW = NC * NS
BLK = 128
BLKS_PER_W = (NTOK // BLK) // NW
B_PER_W = B_TOTAL // NW
N_UNITS = BLKS_PER_W * NSEQ


@functools.partial(
    pl.kernel,
    mesh=plsc.VectorSubcoreMesh(core_axis_name="c", subcore_axis_name="s"),
    out_type=jax.ShapeDtypeStruct((NSEQ, 8, NTOK // BLK, 8, BLK), jnp.float32),
    scratch_types=[
        pltpu.VMEM((B_PER_W,), jnp.int32),
        pltpu.VMEM((BLK, DMODEL), jnp.float32),
        pltpu.VMEM((BLK, DMODEL), jnp.float32),
        pltpu.VMEM((8, 8, BLK), jnp.float32),
        pltpu.VMEM((8, 8, BLK), jnp.float32),
        pltpu.VMEM((BLK,), jnp.int32),
        pltpu.VMEM((BLK,), jnp.int32),
        pltpu.SemaphoreType.DMA,
        pltpu.SemaphoreType.DMA,
        pltpu.SemaphoreType.DMA,
        pltpu.SemaphoreType.DMA,
    ],
    compiler_params=pltpu.CompilerParams(
        use_tc_tiling_on_sc=False, needs_layout_passes=False),
)
def _embed_sc(idx_hbm, table_hbm, out_hbm, idx_all,
              rows0, rows1, t0, t1, gx0, gx1, sg0, sg1, sw0, sw1):
    rows = (rows0, rows1)
    tb = (t0, t1)
    gx = (gx0, gx1)
    sg = (sg0, sg1)
    sw = (sw0, sw1)
    wid = lax.axis_index("s") * NC + lax.axis_index("c")
    base_blk = wid * BLKS_PER_W

    pltpu.sync_copy(idx_hbm.at[pl.ds(wid * B_PER_W, B_PER_W)], idx_all)

    lane = jax.lax.iota(jnp.int32, 16)
    lane50 = lane * NSEQ

    def build_gidx(b, u):
        s = u // BLKS_PER_W
        lb = u % BLKS_PER_W
        base_p = lb * (BLK * NSEQ) + s
        for g in range(8):
            vals = plsc.load_gather(idx_all, [lane50 + (base_p + g * 16 * NSEQ)])
            gx[b][pl.ds(g * 16, 16)] = vals

    def fire_gather(b):
        pltpu.async_copy(table_hbm.at[gx[b]], rows[b], sg[b])

    def wait_gather(b):
        pltpu.make_async_copy(table_hbm.at[gx[b]], rows[b], sg[b]).wait()

    def transpose(b):
        @plsc.parallel_loop(0, 16, unroll=2)
        def r_body(r):
            rot = (lane + r) & 15
            rot8 = rot >> 3
            rotm8 = rot & 7
            for k in range(4):
                dvec = rot + k * 16
                i0 = rot8 + 2 * k
                for g in range(8):
                    jvec = lane + g * 16
                    v = plsc.load_gather(rows[b], [jvec, dvec])
                    plsc.store_scatter(tb[b], [i0, rotm8, jvec], v)

    def fire_wb(b, u):
        s = u // BLKS_PER_W
        bb = base_blk + u % BLKS_PER_W
        pltpu.async_copy(tb[b], out_hbm.at[s, :, bb], sw[b])

    def wait_wb(b):
        pltpu.make_async_copy(tb[b], out_hbm.at[0, :, 0], sw[b]).wait()

    build_gidx(0, 0)
    fire_gather(0)
    build_gidx(1, 1)
    fire_gather(1)
    for u0 in (0, 1):
        b = u0 % 2
        wait_gather(b)
        transpose(b)
        fire_wb(b, u0)
        build_gidx(b, u0 + 2)
        fire_gather(b)

    def outer(k, carry):
        for i in range(2):
            u = 2 * k + i
            b = i
            wait_gather(b)
            wait_wb(b)
            transpose(b)
            fire_wb(b, u)
            build_gidx(b, u + 2)
            fire_gather(b)
        return carry

    lax.fori_loop(1, N_UNITS // 2 - 1, outer, 0)

    for u0 in (N_UNITS - 2, N_UNITS - 1):
        b = u0 % 2
        wait_gather(b)
        wait_wb(b)
        transpose(b)
        fire_wb(b, u0)
    wait_wb(0)
    wait_wb(1)


def kernel(indices, table):
    idx_flat = indices.reshape(-1).astype(jnp.int32)
    out5 = _embed_sc(idx_flat, table)
    return out5.transpose(2, 4, 0, 1, 3).reshape(NTOK, NSEQ, DMODEL)
